# baseline (speedup 1.0000x reference)
import jax
import jax.numpy as jnp
from jax import lax
from jax.experimental import pallas as pl
from jax.experimental.pallas import tpu as pltpu

S = 2048
D = 768
R = 192
RP = 256
NI = 8
NP = 32
K = 3
NC = 3
TB = 256


def _body(x_ref, wr_ref, qin_ref, kin_ref, vin_ref, pn_ref,
          q_ref, k_ref, v_ref, inbf_ref):
    t = pl.program_id(0)

    @pl.when(t == 0)
    def _stage():
        inbf_ref[...] = jnp.zeros((D, NC * NI * RP), jnp.bfloat16)
        for c, ir in enumerate((qin_ref, kin_ref, vin_ref)):
            for n in range(NI):
                base = (c * NI + n) * RP
                inbf_ref[:, base:base + R] = ir[n].astype(jnp.bfloat16)

    x = x_ref[...]
    scores = lax.dot_general(x, wr_ref[...], (((1,), (0,)), ((), ())),
                             preferred_element_type=jnp.float32)
    si = scores[:, :NI]
    sp = scores[:, NI:]
    si = si - jnp.max(si, axis=-1, keepdims=True)
    e = jnp.exp(si)
    w = e / jnp.sum(e, axis=-1, keepdims=True)

    proj = lax.dot_general(x.astype(jnp.bfloat16), inbf_ref[...],
                           (((1,), (0,)), ((), ())),
                           preferred_element_type=jnp.float32)
    xrs = []
    for c in range(NC):
        xr = w[:, 0:1] * proj[:, c * NI * RP:c * NI * RP + RP]
        for n in range(1, NI):
            base = (c * NI + n) * RP
            xr = xr + w[:, n:n + 1] * proj[:, base:base + RP]
        xrs.append(xr[:, :R])

    pn_ns = []
    for c in range(NC):
        blk = pn_ref[c]
        nrm = lax.rsqrt(jnp.sum(blk * blk, axis=-1, keepdims=True) + 1e-8)
        pn_ns.append(blk * nrm)

    iota = lax.broadcasted_iota(jnp.int32, (TB, NP), 1)
    for _ in range(K):
        m = jnp.max(sp, axis=-1, keepdims=True)
        cand = jnp.where(sp == m, iota, NP)
        amin = jnp.min(cand, axis=-1, keepdims=True)
        oh = (iota == amin)
        ohf = oh.astype(jnp.float32)
        for c in range(NC):
            sel = lax.dot_general(ohf, pn_ns[c], (((1,), (0,)), ((), ())),
                                  preferred_element_type=jnp.float32)
            vtx = jnp.sum(xrs[c] * sel, axis=-1, keepdims=True)
            xrs[c] = xrs[c] - 2.0 * sel * vtx
        sp = jnp.where(oh, -jnp.inf, sp)

    q_ref[...] = xrs[0]
    k_ref[...] = xrs[1]
    v_ref[...] = xrs[2]


def kernel(x, Wi, Wp, q_in, q_pn, k_in, k_pn, v_in, v_pn):
    x2 = x.reshape(S, D)
    wr = jnp.concatenate([Wi.T, Wp.T], axis=1)
    pnstk = jnp.stack([q_pn, k_pn, v_pn])
    full = lambda shape: pl.BlockSpec(shape, lambda t: tuple(0 for _ in shape))
    q, k, v = pl.pallas_call(
        _body,
        grid=(S // TB,),
        in_specs=[
            pl.BlockSpec((TB, D), lambda t: (t, 0)),
            full((D, NI + NP)),
            full((NI, D, R)),
            full((NI, D, R)),
            full((NI, D, R)),
            full((NC, NP, R)),
        ],
        out_specs=[
            pl.BlockSpec((TB, R), lambda t: (t, 0)),
            pl.BlockSpec((TB, R), lambda t: (t, 0)),
            pl.BlockSpec((TB, R), lambda t: (t, 0)),
        ],
        out_shape=[jax.ShapeDtypeStruct((S, R), jnp.float32)] * 3,
        scratch_shapes=[pltpu.VMEM((D, NC * NI * RP), jnp.bfloat16)],
    )(x2, wr, q_in, k_in, v_in, pnstk)
    return (q.reshape(1, S, R), k.reshape(1, S, R), v.reshape(1, S, R))
